# baseline (speedup 1.0000x reference)
import functools

import jax
import jax.numpy as jnp
from jax import lax
from jax.experimental import pallas as pl
from jax.experimental.pallas import tpu as pltpu
from jax.experimental.pallas import tpu_sc as plsc

VOCAB = 100000
D_MODEL = 64
BATCH = 1024

VOCAB_TILE = 4096
REPACK_TILE = 4096
REPACK_GRID = 13
SPLIT = REPACK_TILE * REPACK_GRID
LANES = 128


def _repack_body(lo_ref, hi_ref, out_ref):
    r = lax.broadcasted_iota(jnp.int32, (D_MODEL, D_MODEL), 0)
    c = lax.broadcasted_iota(jnp.int32, (D_MODEL, D_MODEL), 1)
    eye = (r == c).astype(jnp.float32)
    dims = (((0,), (0,)), ((), ()))
    out_ref[:, :D_MODEL] = lax.dot_general(
        lo_ref[...], eye, dims, preferred_element_type=jnp.float32
    )
    out_ref[:, D_MODEL:] = lax.dot_general(
        hi_ref[...], eye, dims, preferred_element_type=jnp.float32
    )


def _repack(tabT):
    return pl.pallas_call(
        _repack_body,
        grid=(REPACK_GRID,),
        in_specs=[
            pl.BlockSpec((D_MODEL, REPACK_TILE), lambda i: (0, i)),
            pl.BlockSpec(
                (D_MODEL, REPACK_TILE),
                lambda i: (0, jnp.minimum(i + REPACK_GRID, VOCAB // REPACK_TILE)),
            ),
        ],
        out_specs=pl.BlockSpec((REPACK_TILE, LANES), lambda i: (i, 0)),
        out_shape=jax.ShapeDtypeStruct((SPLIT, LANES), jnp.float32),
    )(tabT, tabT)


@functools.cache
def _gather_kernel(num_cores: int, num_subcores: int):
    nw = num_cores * num_subcores
    b_per_w = BATCH // nw
    mesh = plsc.VectorSubcoreMesh(core_axis_name="c", subcore_axis_name="s")

    @functools.partial(
        pl.kernel,
        mesh=mesh,
        out_type=jax.ShapeDtypeStruct((BATCH, LANES), jnp.float32),
        scratch_types=[
            pltpu.VMEM((b_per_w,), jnp.int32),
            pltpu.VMEM((b_per_w,), jnp.int32),
            pltpu.VMEM((b_per_w, LANES), jnp.float32),
            pltpu.SemaphoreType.DMA,
        ],
    )
    def gather(table_hbm, idx_hbm, out_hbm, idx_v, line_v, rows_v, sem):
        wid = lax.axis_index("s") * num_cores + lax.axis_index("c")
        base = wid * b_per_w
        pltpu.sync_copy(idx_hbm.at[pl.ds(base, b_per_w)], idx_v)
        for c in range(b_per_w // 16):
            sl = pl.ds(c * 16, 16)
            v = idx_v[sl]
            line_v[sl] = jnp.where(v >= SPLIT, v - SPLIT, v)
        pltpu.async_copy(table_hbm.at[line_v], rows_v, sem).wait()
        pltpu.sync_copy(rows_v, out_hbm.at[pl.ds(base, b_per_w)])

    return gather


def _matmul_body(pair_ref, sel_ref, wt_ref, out_ref):
    pair = pair_ref[...]
    sel = sel_ref[...]
    emb = jnp.where(sel > 0, pair[:, D_MODEL:], pair[:, :D_MODEL])
    out_ref[...] = lax.dot_general(
        wt_ref[...],
        emb,
        (((0,), (1,)), ((), ())),
        preferred_element_type=jnp.float32,
    )


def _projection(pair_emb, sel, Wt):
    grid = pl.cdiv(VOCAB, VOCAB_TILE)
    outT = pl.pallas_call(
        _matmul_body,
        grid=(grid,),
        in_specs=[
            pl.BlockSpec((BATCH, LANES), lambda i: (0, 0)),
            pl.BlockSpec((BATCH, 1), lambda i: (0, 0)),
            pl.BlockSpec((D_MODEL, VOCAB_TILE), lambda i: (0, i)),
        ],
        out_specs=pl.BlockSpec((VOCAB_TILE, BATCH), lambda i: (i, 0)),
        out_shape=jax.ShapeDtypeStruct((VOCAB, BATCH), jnp.float32),
    )(pair_emb, sel, Wt)
    return outT.T


def kernel(indices, table, W):
    info = plsc.get_sparse_core_info()
    gather = _gather_kernel(info.num_cores, info.num_subcores)
    table_pairs = _repack(table.T)
    pair_emb = gather(table_pairs, indices)
    sel = (indices >= SPLIT).astype(jnp.int32).reshape(BATCH, 1)
    return _projection(pair_emb, sel, W.T)


# speedup vs baseline: 1.0045x; 1.0045x over previous
import functools

import jax
import jax.numpy as jnp
from jax import lax
from jax.experimental import pallas as pl
from jax.experimental.pallas import tpu as pltpu
from jax.experimental.pallas import tpu_sc as plsc

VOCAB = 100000
D_MODEL = 64
BATCH = 1024

VOCAB_TILE = 4096
REPACK_TILE = 8192
REPACK_GRID = 7
SPLIT = REPACK_TILE * REPACK_GRID
LANES = 128


def _repack_body(lo_ref, hi_ref, out_ref):
    r = lax.broadcasted_iota(jnp.int32, (D_MODEL, D_MODEL), 0)
    c = lax.broadcasted_iota(jnp.int32, (D_MODEL, D_MODEL), 1)
    eye = (r == c).astype(jnp.float32)
    dims = (((0,), (0,)), ((), ()))
    out_ref[:, :D_MODEL] = lax.dot_general(
        lo_ref[...], eye, dims, preferred_element_type=jnp.float32
    )
    out_ref[:, D_MODEL:] = lax.dot_general(
        hi_ref[...], eye, dims, preferred_element_type=jnp.float32
    )


def _repack(tabT):
    return pl.pallas_call(
        _repack_body,
        grid=(REPACK_GRID,),
        in_specs=[
            pl.BlockSpec((D_MODEL, REPACK_TILE), lambda i: (0, i)),
            pl.BlockSpec(
                (D_MODEL, REPACK_TILE),
                lambda i: (0, jnp.minimum(i + REPACK_GRID, VOCAB // REPACK_TILE)),
            ),
        ],
        out_specs=pl.BlockSpec((REPACK_TILE, LANES), lambda i: (i, 0)),
        out_shape=jax.ShapeDtypeStruct((SPLIT, LANES), jnp.float32),
    )(tabT, tabT)


@functools.cache
def _gather_kernel(num_cores: int, num_subcores: int):
    nw = num_cores * num_subcores
    b_per_w = BATCH // nw
    mesh = plsc.VectorSubcoreMesh(core_axis_name="c", subcore_axis_name="s")

    @functools.partial(
        pl.kernel,
        mesh=mesh,
        out_type=jax.ShapeDtypeStruct((BATCH, LANES), jnp.float32),
        scratch_types=[
            pltpu.VMEM((b_per_w,), jnp.int32),
            pltpu.VMEM((b_per_w,), jnp.int32),
            pltpu.VMEM((b_per_w, LANES), jnp.float32),
            pltpu.SemaphoreType.DMA,
        ],
    )
    def gather(table_hbm, idx_hbm, out_hbm, idx_v, line_v, rows_v, sem):
        wid = lax.axis_index("s") * num_cores + lax.axis_index("c")
        base = wid * b_per_w
        pltpu.sync_copy(idx_hbm.at[pl.ds(base, b_per_w)], idx_v)
        for c in range(b_per_w // 16):
            sl = pl.ds(c * 16, 16)
            v = idx_v[sl]
            line_v[sl] = jnp.where(v >= SPLIT, v - SPLIT, v)
        pltpu.async_copy(table_hbm.at[line_v], rows_v, sem).wait()
        pltpu.sync_copy(rows_v, out_hbm.at[pl.ds(base, b_per_w)])

    return gather


def _matmul_body(pair_ref, sel_ref, wt_ref, out_ref):
    pair = pair_ref[...]
    sel = sel_ref[...]
    emb = jnp.where(sel > 0, pair[:, D_MODEL:], pair[:, :D_MODEL])
    out_ref[...] = lax.dot_general(
        wt_ref[...],
        emb,
        (((0,), (1,)), ((), ())),
        preferred_element_type=jnp.float32,
    )


def _projection(pair_emb, sel, Wt):
    grid = pl.cdiv(VOCAB, VOCAB_TILE)
    outT = pl.pallas_call(
        _matmul_body,
        grid=(grid,),
        in_specs=[
            pl.BlockSpec((BATCH, LANES), lambda i: (0, 0)),
            pl.BlockSpec((BATCH, 1), lambda i: (0, 0)),
            pl.BlockSpec((D_MODEL, VOCAB_TILE), lambda i: (0, i)),
        ],
        out_specs=pl.BlockSpec((VOCAB_TILE, BATCH), lambda i: (i, 0)),
        out_shape=jax.ShapeDtypeStruct((VOCAB, BATCH), jnp.float32),
    )(pair_emb, sel, Wt)
    return outT.T


def kernel(indices, table, W):
    info = plsc.get_sparse_core_info()
    gather = _gather_kernel(info.num_cores, info.num_subcores)
    table_pairs = _repack(table.T)
    pair_emb = gather(table_pairs, indices)
    sel = (indices >= SPLIT).astype(jnp.int32).reshape(BATCH, 1)
    return _projection(pair_emb, sel, W.T)
